# baseline (speedup 1.0000x reference)
import functools

import jax
import jax.numpy as jnp
from jax.experimental import pallas as pl
from jax.experimental.pallas import tpu as pltpu

H = 512
W = 512
NPIX = float(H * W)


def _shift(x, axis, shift):
    return jnp.roll(x, shift, axis=axis)


def _loss_kernel(p_ref, t_ref, o_ref):
    import numpy as np

    p0 = p_ref[0]
    best = p0
    cls = jnp.zeros((H, W), jnp.int32)
    for c in range(1, 5):
        pc = p_ref[c]
        m = pc > best
        best = jnp.where(m, pc, best)
        cls = jnp.where(m, c, cls)

    ppo = jnp.where(cls > 0, best, 0.0)

    tgt = t_ref[...]

    sig = jax.nn.sigmoid(ppo)
    gterm = jnp.maximum(ppo, 0.0) + jnp.log1p(jnp.exp(-jnp.abs(ppo)))

    LOG2 = 0.6931471805599453
    L1 = 0.3132616875182228
    SIG1 = 0.7310585786300049

    colmax = jnp.max(cls, axis=0, keepdims=True)
    colzero = colmax == 0
    t0 = tgt == 0
    Z = jnp.sum(colzero.astype(jnp.float32))
    q = jnp.sum(jnp.where(colzero & t0, 1.0, 0.0))
    n0t = jnp.sum(t0.astype(jnp.float32))
    bce0 = ((NPIX - H * Z) * LOG2 + H * Z * (1.0 + L1) - q) / NPIX
    inter0 = SIG1 * q + 0.5 * (n0t - q)
    sumP0 = SIG1 * H * Z + 0.5 * (NPIX - H * Z)
    dice0 = 1.0 - (2.0 * inter0 + 1.0) / (sumP0 + n0t + 1.0)
    res = bce0 + dice0

    n_c, A_c, S_c = [], [], []
    n_ct, B_ct, S_ct = [], [], []
    mt = [tgt == t for t in range(5)]
    for c in range(5):
        mc = cls == c
        A_c.append(jnp.sum(jnp.where(mc, gterm, 0.0)))
        nr, Br, Sr = [], [], []
        for t in range(5):
            mct = mc & mt[t]
            nr.append(jnp.sum(mct.astype(jnp.int32)))
            Br.append(jnp.sum(jnp.where(mct, ppo, 0.0)))
            Sr.append(jnp.sum(jnp.where(mct, sig, 0.0)))
        n_ct.append(nr)
        B_ct.append(Br)
        S_ct.append(Sr)
        n_c.append(nr[0] + nr[1] + nr[2] + nr[3] + nr[4])
        S_c.append(Sr[0] + Sr[1] + Sr[2] + Sr[3] + Sr[4])

    row = jax.lax.broadcasted_iota(jnp.int32, (H, W), 0)
    col = jax.lax.broadcasted_iota(jnp.int32, (H, W), 1)
    idx = row * W + col + 1
    lab0 = jnp.where(cls > 0, idx, 0)

    adj1 = (jnp.where(row == 0, -1, _shift(cls, 0, 1)) == cls)
    adj2 = (jnp.where(row == H - 1, -1, _shift(cls, 0, -1)) == cls)
    adj3 = (jnp.where(col == 0, -1, _shift(cls, 1, 1)) == cls)
    adj4 = (jnp.where(col == W - 1, -1, _shift(cls, 1, -1)) == cls)

    def step(lab):
        m = lab
        m = jnp.maximum(m, jnp.where(adj1, _shift(lab, 0, 1), 0))
        m = jnp.maximum(m, jnp.where(adj2, _shift(lab, 0, -1), 0))
        m = jnp.maximum(m, jnp.where(adj3, _shift(lab, 1, 1), 0))
        m = jnp.maximum(m, jnp.where(adj4, _shift(lab, 1, -1), 0))
        return m

    lab0 = step(step(step(step(lab0))))
    lab0 = step(step(step(step(lab0))))
    lab0 = step(step(step(step(lab0))))
    lab0 = step(step(lab0))

    def body(carry):
        lab, _ = carry
        new = step(lab)
        return new, jnp.any(new != lab)

    def cond(carry):
        return carry[1]

    lab, _ = jax.lax.while_loop(cond, body, (lab0, jnp.bool_(True)))

    ncc = []
    roots = lab == idx
    for v in range(1, 5):
        ncc.append(jnp.sum(((cls == v) & roots).astype(jnp.int32)))

    last_i = jnp.int32(1)
    a = [jnp.float32(0.0)] * 5
    for v in range(1, 5):
        present = n_c[v] > 0
        n_nz = ncc[v - 1]
        n_uniq = n_nz + (n_c[v] < H * W).astype(jnp.int32)
        s_v = (n_nz * last_i).astype(jnp.float32)
        a = [(a[c] + (1.0 if c == v else 0.0)) + s_v for c in range(5)]
        last_i = last_i + jnp.where(present, n_uniq, 0)

    for t in range(1, 5):
        n = n_ct[0][t] + n_ct[1][t] + n_ct[2][t] + n_ct[3][t] + n_ct[4][t]
        k = (n - 1) // 2
        med = jnp.float32(jnp.inf)
        for c in range(5):
            cum = jnp.int32(0)
            for c2 in range(5):
                cum = cum + jnp.where(a[c2] <= a[c], n_ct[c2][t], 0)
            med = jnp.minimum(med, jnp.where(cum >= k + 1, a[c],
                                             jnp.float32(jnp.inf)))
        nM = jnp.float32(0.0)
        sumA = jnp.float32(0.0)
        sumB_Mt = jnp.float32(0.0)
        sumS_M = jnp.float32(0.0)
        sumS_Mt = jnp.float32(0.0)
        n_Mt = jnp.float32(0.0)
        sumB_t = jnp.float32(0.0)
        for c in range(5):
            match = a[c] == med
            nM = nM + jnp.where(match, n_c[c].astype(jnp.float32), 0.0)
            sumA = sumA + jnp.where(match, A_c[c], 0.0)
            sumB_Mt = sumB_Mt + jnp.where(match, B_ct[c][t], 0.0)
            sumS_M = sumS_M + jnp.where(match, S_c[c], 0.0)
            sumS_Mt = sumS_Mt + jnp.where(match, S_ct[c][t], 0.0)
            n_Mt = n_Mt + jnp.where(match, n_ct[c][t].astype(jnp.float32), 0.0)
            sumB_t = sumB_t + B_ct[c][t]
        nf = n.astype(jnp.float32)
        bce = (sumA + (NPIX - nM) * LOG2 - sumB_Mt) / NPIX
        inter = sumS_Mt + 0.5 * (nf - n_Mt)
        sumP = sumS_M + 0.5 * (NPIX - nM)
        dice = 1.0 - (2.0 * inter + 1.0) / (sumP + nf + 1.0)
        extra = (sumB_t - sumB_Mt) / nf
        contrib = bce + dice + extra
        res = res + jnp.where(n > 0, contrib, 0.0)

    n_t_total = jnp.int32(0)
    for t in range(5):
        cnt_t = (n_ct[0][t] + n_ct[1][t] + n_ct[2][t] + n_ct[3][t]
                 + n_ct[4][t])
        n_t_total = n_t_total + (cnt_t > 0).astype(jnp.int32)

    o_ref[0, 0] = res / (n_t_total * 2 + 1).astype(jnp.float32)


@functools.partial(jax.jit, static_argnames=("interpret",))
def _run(pred_out, target_mask, interpret=False):
    p = pred_out.reshape(5, H, W)
    tgt = target_mask.reshape(H, W)
    out = pl.pallas_call(
        _loss_kernel,
        out_shape=jax.ShapeDtypeStruct((1, 1), jnp.float32),
        in_specs=[
            pl.BlockSpec(memory_space=pltpu.VMEM),
            pl.BlockSpec(memory_space=pltpu.VMEM),
        ],
        out_specs=pl.BlockSpec(memory_space=pltpu.SMEM),
        interpret=interpret,
    )(p, tgt)
    return out[0, 0]


def kernel(pred_out, target_mask):
    return _run(pred_out, target_mask)


# speedup vs baseline: 1.0007x; 1.0007x over previous
import jax
import jax.numpy as jnp
from jax.experimental import pallas as pl
from jax.experimental.pallas import tpu as pltpu

H = 512
W = 512
NPIX = float(H * W)


def _shift(x, axis, shift):
    return jnp.roll(x, shift, axis=axis)


def _loss_kernel(p_ref, t_ref, o_ref):
    p0 = p_ref[0]
    best = p0
    cls = jnp.zeros((H, W), jnp.int32)
    for c in range(1, 5):
        pc = p_ref[c]
        m = pc > best
        best = jnp.where(m, pc, best)
        cls = jnp.where(m, c, cls)

    ppo = jnp.where(cls > 0, best, 0.0)

    tgt = t_ref[...]

    sig = jax.nn.sigmoid(ppo)
    gterm = jnp.maximum(ppo, 0.0) + jnp.log1p(jnp.exp(-jnp.abs(ppo)))

    LOG2 = 0.6931471805599453
    L1 = 0.3132616875182228
    SIG1 = 0.7310585786300049

    colmax = jnp.max(cls, axis=0, keepdims=True)
    colzero = colmax == 0
    t0 = tgt == 0
    Z = jnp.sum(colzero.astype(jnp.float32))
    q = jnp.sum(jnp.where(colzero & t0, 1.0, 0.0))
    n0t = jnp.sum(t0.astype(jnp.float32))
    bce0 = ((NPIX - H * Z) * LOG2 + H * Z * (1.0 + L1) - q) / NPIX
    inter0 = SIG1 * q + 0.5 * (n0t - q)
    sumP0 = SIG1 * H * Z + 0.5 * (NPIX - H * Z)
    dice0 = 1.0 - (2.0 * inter0 + 1.0) / (sumP0 + n0t + 1.0)
    res = bce0 + dice0

    n_c, A_c, S_c = [], [], []
    n_ct, B_ct, S_ct = [], [], []
    mt = [tgt == t for t in range(5)]
    for c in range(5):
        mc = cls == c
        A_c.append(jnp.sum(jnp.where(mc, gterm, 0.0)))
        nr, Br, Sr = [], [], []
        for t in range(5):
            mct = mc & mt[t]
            nr.append(jnp.sum(mct.astype(jnp.int32)))
            Br.append(jnp.sum(jnp.where(mct, ppo, 0.0)))
            Sr.append(jnp.sum(jnp.where(mct, sig, 0.0)))
        n_ct.append(nr)
        B_ct.append(Br)
        S_ct.append(Sr)
        n_c.append(nr[0] + nr[1] + nr[2] + nr[3] + nr[4])
        S_c.append(Sr[0] + Sr[1] + Sr[2] + Sr[3] + Sr[4])

    row = jax.lax.broadcasted_iota(jnp.int32, (H, W), 0)
    col = jax.lax.broadcasted_iota(jnp.int32, (H, W), 1)
    idx = row * W + col + 1
    lab0 = jnp.where(cls > 0, idx, 0)

    adj1 = (jnp.where(row == 0, -1, _shift(cls, 0, 1)) == cls)
    adj2 = (jnp.where(row == H - 1, -1, _shift(cls, 0, -1)) == cls)
    adj3 = (jnp.where(col == 0, -1, _shift(cls, 1, 1)) == cls)
    adj4 = (jnp.where(col == W - 1, -1, _shift(cls, 1, -1)) == cls)

    def step(lab):
        m = lab
        m = jnp.maximum(m, jnp.where(adj1, _shift(lab, 0, 1), 0))
        m = jnp.maximum(m, jnp.where(adj2, _shift(lab, 0, -1), 0))
        m = jnp.maximum(m, jnp.where(adj3, _shift(lab, 1, 1), 0))
        m = jnp.maximum(m, jnp.where(adj4, _shift(lab, 1, -1), 0))
        return m

    lab0 = step(step(step(step(lab0))))
    lab0 = step(step(step(step(lab0))))
    lab0 = step(step(step(step(lab0))))
    lab0 = step(step(lab0))

    def body(carry):
        lab, _ = carry
        new = step(lab)
        return new, jnp.any(new != lab)

    def cond(carry):
        return carry[1]

    lab, _ = jax.lax.while_loop(cond, body, (lab0, jnp.bool_(True)))

    ncc = []
    roots = lab == idx
    for v in range(1, 5):
        ncc.append(jnp.sum(((cls == v) & roots).astype(jnp.int32)))

    last_i = jnp.int32(1)
    a = [jnp.float32(0.0)] * 5
    for v in range(1, 5):
        present = n_c[v] > 0
        n_nz = ncc[v - 1]
        n_uniq = n_nz + (n_c[v] < H * W).astype(jnp.int32)
        s_v = (n_nz * last_i).astype(jnp.float32)
        a = [(a[c] + (1.0 if c == v else 0.0)) + s_v for c in range(5)]
        last_i = last_i + jnp.where(present, n_uniq, 0)

    for t in range(1, 5):
        n = n_ct[0][t] + n_ct[1][t] + n_ct[2][t] + n_ct[3][t] + n_ct[4][t]
        k = (n - 1) // 2
        med = jnp.float32(jnp.inf)
        for c in range(5):
            cum = jnp.int32(0)
            for c2 in range(5):
                cum = cum + jnp.where(a[c2] <= a[c], n_ct[c2][t], 0)
            med = jnp.minimum(med, jnp.where(cum >= k + 1, a[c],
                                             jnp.float32(jnp.inf)))
        nM = jnp.float32(0.0)
        sumA = jnp.float32(0.0)
        sumB_Mt = jnp.float32(0.0)
        sumS_M = jnp.float32(0.0)
        sumS_Mt = jnp.float32(0.0)
        n_Mt = jnp.float32(0.0)
        sumB_t = jnp.float32(0.0)
        for c in range(5):
            match = a[c] == med
            nM = nM + jnp.where(match, n_c[c].astype(jnp.float32), 0.0)
            sumA = sumA + jnp.where(match, A_c[c], 0.0)
            sumB_Mt = sumB_Mt + jnp.where(match, B_ct[c][t], 0.0)
            sumS_M = sumS_M + jnp.where(match, S_c[c], 0.0)
            sumS_Mt = sumS_Mt + jnp.where(match, S_ct[c][t], 0.0)
            n_Mt = n_Mt + jnp.where(match, n_ct[c][t].astype(jnp.float32), 0.0)
            sumB_t = sumB_t + B_ct[c][t]
        nf = n.astype(jnp.float32)
        bce = (sumA + (NPIX - nM) * LOG2 - sumB_Mt) / NPIX
        inter = sumS_Mt + 0.5 * (nf - n_Mt)
        sumP = sumS_M + 0.5 * (NPIX - nM)
        dice = 1.0 - (2.0 * inter + 1.0) / (sumP + nf + 1.0)
        extra = (sumB_t - sumB_Mt) / nf
        contrib = bce + dice + extra
        res = res + jnp.where(n > 0, contrib, 0.0)

    n_t_total = jnp.int32(0)
    for t in range(5):
        cnt_t = (n_ct[0][t] + n_ct[1][t] + n_ct[2][t] + n_ct[3][t]
                 + n_ct[4][t])
        n_t_total = n_t_total + (cnt_t > 0).astype(jnp.int32)

    o_ref[0, 0] = res / (n_t_total * 2 + 1).astype(jnp.float32)


@jax.jit
def _run(pred_out, target_mask):
    p = pred_out.reshape(5, H, W)
    tgt = target_mask.reshape(H, W)
    out = pl.pallas_call(
        _loss_kernel,
        out_shape=jax.ShapeDtypeStruct((1, 1), jnp.float32),
        in_specs=[
            pl.BlockSpec(memory_space=pltpu.VMEM),
            pl.BlockSpec(memory_space=pltpu.VMEM),
        ],
        out_specs=pl.BlockSpec(memory_space=pltpu.SMEM),
    )(p, tgt)
    return out[0, 0]


def kernel(pred_out, target_mask):
    return _run(pred_out, target_mask)
